# baseline (speedup 1.0000x reference)
import functools

import jax
import jax.numpy as jnp
from jax import lax
from jax.experimental import pallas as pl
from jax.experimental.pallas import tpu as pltpu
from jax.experimental.pallas import tpu_sc as plsc

S = 4096
D = 2048
E = 16
CAP = 256

BT1 = 128
NB1 = S // BT1
BT2 = 256
NB2 = S // BT2
NW = 32
TPT = S // NW
LANES = 16


def _tc1_body(x_ref, wt_ref, exp_ref, gate_ref, cntpre_ref, counts_ref,
              laux_ref, me_acc):
    g = pl.program_id(0)
    logits = jnp.dot(x_ref[...], wt_ref[...],
                     preferred_element_type=jnp.float32)
    lmax = jnp.max(logits, axis=1, keepdims=True)
    ex = jnp.exp(logits - lmax)
    den = jnp.sum(ex, axis=1, keepdims=True)
    gate_ref[...] = 1.0 / den

    iota_e = lax.broadcasted_iota(jnp.int32, (BT1, E), 1)
    eq = logits == lmax
    expert = jnp.min(jnp.where(eq, iota_e, E), axis=1, keepdims=True)
    exp_ref[...] = expert

    @pl.when(g == 0)
    def _():
        counts_ref[...] = jnp.zeros_like(counts_ref)
        me_acc[...] = jnp.zeros_like(me_acc)

    cntpre_ref[...] = counts_ref[...].reshape(1, 1, E)

    onehot = (iota_e == expert).astype(jnp.int32)
    counts_ref[...] += jnp.sum(onehot, axis=0, keepdims=True)
    me_acc[...] += jnp.sum(ex / den, axis=0, keepdims=True)

    @pl.when(g == NB1 - 1)
    def _():
        laux = jnp.sum(me_acc[...] * counts_ref[...].astype(jnp.float32))
        laux_ref[...] = jnp.full((1, 1), laux * (E / (S * S)), jnp.float32)


def _tc1(x, wt):
    return pl.pallas_call(
        _tc1_body,
        grid=(NB1,),
        in_specs=[
            pl.BlockSpec((BT1, D), lambda g: (g, 0)),
            pl.BlockSpec((D, E), lambda g: (0, 0)),
        ],
        out_specs=[
            pl.BlockSpec((BT1, 1), lambda g: (g, 0)),
            pl.BlockSpec((BT1, 1), lambda g: (g, 0)),
            pl.BlockSpec((1, 1, E), lambda g: (g, 0, 0)),
            pl.BlockSpec((1, E), lambda g: (0, 0)),
            pl.BlockSpec((1, 1), lambda g: (0, 0)),
        ],
        out_shape=[
            jax.ShapeDtypeStruct((S, 1), jnp.int32),
            jax.ShapeDtypeStruct((S, 1), jnp.float32),
            jax.ShapeDtypeStruct((NB1, 1, E), jnp.int32),
            jax.ShapeDtypeStruct((1, E), jnp.int32),
            jax.ShapeDtypeStruct((1, 1), jnp.float32),
        ],
        scratch_shapes=[pltpu.VMEM((1, E), jnp.float32)],
        compiler_params=pltpu.CompilerParams(
            dimension_semantics=("arbitrary",)),
    )(x, wt)


def _gather16(x, idx):
    return lax.gather(
        x, idx[:, None],
        lax.GatherDimensionNumbers(offset_dims=(),
                                   collapsed_slice_dims=(0,),
                                   start_index_map=(0,)),
        slice_sizes=(1,),
        mode=lax.GatherScatterMode.PROMISE_IN_BOUNDS)


def _sc_route_body(exp_hbm, cntpre_hbm, col_hbm, e_v, col_v, cnt_v):
    c = lax.axis_index("c")
    s = lax.axis_index("s")
    wid = c * 16 + s
    base = wid * TPT
    pltpu.sync_copy(exp_hbm.at[pl.ds(base, TPT)], e_v)
    pltpu.sync_copy(cntpre_hbm.at[wid], cnt_v)

    lane = lax.iota(jnp.int32, LANES)
    cnt = cnt_v[...]
    for v in range(TPT // LANES):
        e = e_v[pl.ds(v * LANES, LANES)]
        before = lane * 0
        for k in range(1, LANES):
            idx = (lane - k) & (LANES - 1)
            eqk = 1 - jnp.minimum(jnp.abs(e - _gather16(e, idx)), 1)
            if k > 1:
                gek = jnp.minimum(jnp.maximum(lane - (k - 1), 0), 1)
                before = before + eqk * gek
            else:
                before = before + eqk * jnp.minimum(lane, 1)
        prev = _gather16(cnt, e)
        rank = prev + before
        hist = lane * 0
        for i in range(LANES):
            di = lane - _gather16(e, lane * 0 + i)
            hist = hist + 1 - jnp.minimum(jnp.abs(di), 1)
        cnt = cnt + hist
        valid = 1 - jnp.minimum(jnp.maximum(rank - (CAP - 1), 0), 1)
        col_v[pl.ds(v * LANES, LANES)] = valid * (e * CAP + rank + 1) - 1
    pltpu.sync_copy(col_v, col_hbm.at[pl.ds(base, TPT)])


def _sc_route(expert, cntpre):
    return pl.kernel(
        _sc_route_body,
        mesh=plsc.VectorSubcoreMesh(core_axis_name="c", subcore_axis_name="s"),
        out_type=jax.ShapeDtypeStruct((S,), jnp.int32),
        scratch_types=[
            pltpu.VMEM((TPT,), jnp.int32),
            pltpu.VMEM((TPT,), jnp.int32),
            pltpu.VMEM((E,), jnp.int32),
        ],
    )(expert, cntpre)


def _tc2_body(col_ref, gate_ref, comb_ref, disp_ref):
    iota_c = lax.broadcasted_iota(jnp.int32, (BT2, E * CAP), 1)
    eq = iota_c == col_ref[...]
    comb_ref[...] = jnp.where(eq, gate_ref[...], 0.0)
    disp_ref[...] = eq


def _tc2(col, gate):
    return pl.pallas_call(
        _tc2_body,
        grid=(NB2,),
        in_specs=[
            pl.BlockSpec((BT2, 1), lambda g: (g, 0)),
            pl.BlockSpec((BT2, 1), lambda g: (g, 0)),
        ],
        out_specs=[
            pl.BlockSpec((BT2, E * CAP), lambda g: (g, 0)),
            pl.BlockSpec((BT2, E * CAP), lambda g: (g, 0)),
        ],
        out_shape=[
            jax.ShapeDtypeStruct((S, E * CAP), jnp.float32),
            jax.ShapeDtypeStruct((S, E * CAP), jnp.bool_),
        ],
        compiler_params=pltpu.CompilerParams(
            dimension_semantics=("parallel",)),
    )(col, gate)


def kernel(inputs, W):
    wt = W.T
    expert, gate, cntpre, counts, laux = _tc1(inputs, wt)
    col = _sc_route(expert.reshape(S), cntpre.reshape(NB1, E))
    comb, disp = _tc2(col.reshape(S, 1), gate)
    return (laux.reshape(()),
            comb.reshape(S, E, CAP),
            disp.reshape(S, E, CAP),
            counts.reshape(E))


# speedup vs baseline: 1.5671x; 1.5671x over previous
import functools

import jax
import jax.numpy as jnp
from jax import lax
from jax.experimental import pallas as pl
from jax.experimental.pallas import tpu as pltpu
from jax.experimental.pallas import tpu_sc as plsc

S = 4096
D = 2048
E = 16
CAP = 256

BT1 = 128
NB1 = S // BT1
BT2 = 256
NB2 = S // BT2
NW = 32
TPT = S // NW
LANES = 16


def _tc1_body(x_ref, wt_ref, exp_ref, gate_ref, cntpre_ref, counts_ref,
              laux_ref, me_acc):
    g = pl.program_id(0)
    logits = jnp.dot(x_ref[...], wt_ref[...],
                     preferred_element_type=jnp.float32)
    lmax = jnp.max(logits, axis=1, keepdims=True)
    ex = jnp.exp(logits - lmax)
    den = jnp.sum(ex, axis=1, keepdims=True)
    gate_ref[...] = 1.0 / den

    iota_e = lax.broadcasted_iota(jnp.int32, (BT1, E), 1)
    eq = logits == lmax
    expert = jnp.min(jnp.where(eq, iota_e, E), axis=1, keepdims=True)
    exp_ref[...] = expert

    @pl.when(g == 0)
    def _():
        counts_ref[...] = jnp.zeros_like(counts_ref)
        me_acc[...] = jnp.zeros_like(me_acc)

    cntpre_ref[...] = counts_ref[...].reshape(1, 1, E)

    onehot = (iota_e == expert).astype(jnp.int32)
    counts_ref[...] += jnp.sum(onehot, axis=0, keepdims=True)
    me_acc[...] += jnp.sum(ex / den, axis=0, keepdims=True)

    @pl.when(g == NB1 - 1)
    def _():
        laux = jnp.sum(me_acc[...] * counts_ref[...].astype(jnp.float32))
        laux_ref[...] = jnp.full((1, 1), laux * (E / (S * S)), jnp.float32)


def _tc1(x, wt):
    return pl.pallas_call(
        _tc1_body,
        grid=(NB1,),
        in_specs=[
            pl.BlockSpec((BT1, D), lambda g: (g, 0)),
            pl.BlockSpec((D, E), lambda g: (0, 0)),
        ],
        out_specs=[
            pl.BlockSpec((BT1, 1), lambda g: (g, 0)),
            pl.BlockSpec((BT1, 1), lambda g: (g, 0)),
            pl.BlockSpec((1, 1, E), lambda g: (g, 0, 0)),
            pl.BlockSpec((1, E), lambda g: (0, 0)),
            pl.BlockSpec((1, 1), lambda g: (0, 0)),
        ],
        out_shape=[
            jax.ShapeDtypeStruct((S, 1), jnp.int32),
            jax.ShapeDtypeStruct((S, 1), jnp.float32),
            jax.ShapeDtypeStruct((NB1, 1, E), jnp.int32),
            jax.ShapeDtypeStruct((1, E), jnp.int32),
            jax.ShapeDtypeStruct((1, 1), jnp.float32),
        ],
        scratch_shapes=[pltpu.VMEM((1, E), jnp.float32)],
        compiler_params=pltpu.CompilerParams(
            dimension_semantics=("arbitrary",)),
    )(x, wt)


def _gather16(x, idx):
    return lax.gather(
        x, idx[:, None],
        lax.GatherDimensionNumbers(offset_dims=(),
                                   collapsed_slice_dims=(0,),
                                   start_index_map=(0,)),
        slice_sizes=(1,),
        mode=lax.GatherScatterMode.PROMISE_IN_BOUNDS)


def _sc_route_body(exp_hbm, cntpre_hbm, col_hbm, e_v, col_v, cnt_v):
    c = lax.axis_index("c")
    s = lax.axis_index("s")
    wid = c * 16 + s
    base = wid * TPT
    pltpu.sync_copy(exp_hbm.at[pl.ds(base, TPT)], e_v)
    pltpu.sync_copy(cntpre_hbm.at[wid], cnt_v)

    lane = lax.iota(jnp.int32, LANES)
    cnt = cnt_v[...]
    for v in range(TPT // LANES):
        e = e_v[pl.ds(v * LANES, LANES)]
        before = lane * 0
        for k in range(1, LANES):
            idx = (lane - k) & (LANES - 1)
            eqk = 1 - jnp.minimum(jnp.abs(e - _gather16(e, idx)), 1)
            if k > 1:
                gek = jnp.minimum(jnp.maximum(lane - (k - 1), 0), 1)
                before = before + eqk * gek
            else:
                before = before + eqk * jnp.minimum(lane, 1)
        prev = _gather16(cnt, e)
        rank = prev + before
        hist = lane * 0
        for i in range(LANES):
            di = lane - _gather16(e, lane * 0 + i)
            hist = hist + 1 - jnp.minimum(jnp.abs(di), 1)
        cnt = cnt + hist
        valid = 1 - jnp.minimum(jnp.maximum(rank - (CAP - 1), 0), 1)
        col_v[pl.ds(v * LANES, LANES)] = valid * (e * CAP + rank + 1) - 1
    pltpu.sync_copy(col_v, col_hbm.at[pl.ds(base, TPT)])


def _sc_route(expert, cntpre):
    return pl.kernel(
        _sc_route_body,
        mesh=plsc.VectorSubcoreMesh(core_axis_name="c", subcore_axis_name="s"),
        out_type=jax.ShapeDtypeStruct((S,), jnp.int32),
        scratch_types=[
            pltpu.VMEM((TPT,), jnp.int32),
            pltpu.VMEM((TPT,), jnp.int32),
            pltpu.VMEM((E,), jnp.int32),
        ],
    )(expert, cntpre)


def _tc2_body(col_ref, gate_ref, comb_ref, disp_ref):
    flat = (lax.broadcasted_iota(jnp.int32, (BT2, E, CAP), 1) * CAP
            + lax.broadcasted_iota(jnp.int32, (BT2, E, CAP), 2))
    eq = flat == col_ref[...].reshape(BT2, 1, 1)
    comb_ref[...] = jnp.where(eq, gate_ref[...].reshape(BT2, 1, 1), 0.0)
    disp_ref[...] = eq


def _tc2(col, gate):
    return pl.pallas_call(
        _tc2_body,
        grid=(NB2,),
        in_specs=[
            pl.BlockSpec((BT2, 1), lambda g: (g, 0)),
            pl.BlockSpec((BT2, 1), lambda g: (g, 0)),
        ],
        out_specs=[
            pl.BlockSpec((BT2, E, CAP), lambda g: (g, 0, 0)),
            pl.BlockSpec((BT2, E, CAP), lambda g: (g, 0, 0)),
        ],
        out_shape=[
            jax.ShapeDtypeStruct((S, E, CAP), jnp.float32),
            jax.ShapeDtypeStruct((S, E, CAP), jnp.bool_),
        ],
        compiler_params=pltpu.CompilerParams(
            dimension_semantics=("parallel",)),
    )(col, gate)


def kernel(inputs, W):
    wt = W.T
    expert, gate, cntpre, counts, laux = _tc1(inputs, wt)
    col = _sc_route(expert.reshape(S), cntpre.reshape(NB1, E))
    comb, disp = _tc2(col.reshape(S, 1), gate)
    return (laux.reshape(()), comb, disp, counts.reshape(E))
